# baseline (speedup 1.0000x reference)
import functools

import jax
import jax.numpy as jnp
from jax import lax
from jax.experimental import pallas as pl
from jax.experimental.pallas import tpu as pltpu
from jax.experimental.pallas import tpu_sc as plsc

N = 10000
E = 160000
D = 256

NC = 2
NS = 16
EPS = E // NS
NPAD = 10240
TR = NPAD // 128
RSPAN = 632
CH = 80
NCH = EPS // CH
SUP = 25
NSUP = NCH // SUP
SROWS = 32


def _sc_edge_layer(h2, el2d, er2d, src2d, dst2d):
  mesh = plsc.VectorSubcoreMesh(core_axis_name="c", subcore_axis_name="s")

  @functools.partial(
      pl.kernel,
      out_type=jax.ShapeDtypeStruct((NC, N, 128), jnp.float32),
      mesh=mesh,
      compiler_params=pltpu.CompilerParams(needs_layout_passes=False),
      scratch_types=[
          pltpu.VMEM((TR, 128), jnp.float32),
          pltpu.VMEM((TR, 128), jnp.float32),
          pltpu.VMEM((EPS,), jnp.float32),
          pltpu.VMEM((CH, 128), jnp.float32),
          pltpu.VMEM((SROWS, CH), jnp.int32),
          pltpu.VMEM((SROWS, CH), jnp.int32),
          pltpu.VMEM((CH,), jnp.int32),
          pltpu.VMEM((CH,), jnp.int32),
          pltpu.VMEM((CH,), jnp.int32),
          pltpu.VMEM((TR,), jnp.int32),
          pltpu.SemaphoreType.DMA,
          pltpu.SemaphoreType.DMA,
          pltpu.SemaphoreType.DMA,
          pltpu.SemaphoreType.DMA,
          pltpu.SemaphoreType.DMA,
          pltpu.SemaphoreType.DMA,
          pltpu.VMEM_SHARED((N, 128), jnp.float32),
          pltpu.VMEM_SHARED((TR, 128), jnp.float32),
      ],
  )
  def k(h2_hbm, el_hbm, er_hbm, s2d_hbm, d2d_hbm, out_hbm,
        tab_v, den_v, ee_v, rows_v, src_sup, dst_sup,
        gi0, gi1, gi2, ident_v,
        semg0, semg1, semg2, sems0, sems1, sems2, accum, den_sh):
    c = lax.axis_index("c")
    s = lax.axis_index("s")
    z16 = jnp.zeros((16,), jnp.float32)
    iota16 = lax.iota(jnp.int32, 16)

    def zrows(i, _):
      rows_v[i // 8, pl.ds((i % 8) * 16, 16)] = z16
      return 0
    lax.fori_loop(0, CH * 8, zrows, 0)

    def zden(i, _):
      den_v[i // 8, pl.ds((i % 8) * 16, 16)] = z16
      return 0
    lax.fori_loop(0, TR * 8, zden, 0)

    def mkid(j, _):
      ident_v[pl.ds(j * 16, 16)] = j * 16 + iota16
      return 0
    lax.fori_loop(0, TR // 16, mkid, 0)

    abase = pl.multiple_of((s * (N // NS)) // 8 * 8, 8)
    nfull = RSPAN // CH
    rem = RSPAN - nfull * CH
    for r in range(nfull):
      pltpu.async_copy(rows_v, accum.at[pl.ds(abase + r * CH, CH)], semg0)
    pltpu.async_copy(rows_v.at[pl.ds(0, rem)],
                     accum.at[pl.ds(abase + nfull * CH, rem)], semg1)

    @pl.when(s < TR // 8)
    def _():
      pltpu.async_copy(rows_v.at[pl.ds(0, 8)],
                       den_sh.at[pl.ds(pl.multiple_of(s * 8, 8), 8)], semg2)

    for r in range(nfull):
      pltpu.make_async_copy(
          rows_v, accum.at[pl.ds(abase, CH)], semg0).wait()
    pltpu.make_async_copy(
        rows_v.at[pl.ds(0, rem)], accum.at[pl.ds(abase, rem)], semg1).wait()

    @pl.when(s < TR // 8)
    def _():
      pltpu.make_async_copy(
          rows_v.at[pl.ds(0, 8)], den_sh.at[pl.ds(0, 8)], semg2).wait()

    def sup_base(u):
      r0 = s * NCH + u * SUP
      r0a = pl.multiple_of(r0 // 8 * 8, 8)
      return r0a, r0 - r0a

    pltpu.sync_copy(el_hbm, tab_v)
    for u in range(NSUP):
      r0a, off = sup_base(u)
      pltpu.sync_copy(s2d_hbm.at[pl.ds(r0a, SROWS)], src_sup)

      @plsc.parallel_loop(0, SUP * CH // 16, 1, unroll=4)
      def p1a(i):
        s16 = src_sup[off + i // 5, pl.ds((i % 5) * 16, 16)]
        ee_v[pl.ds(u * SUP * CH + i * 16, 16)] = plsc.load_gather(
            tab_v, [s16 >> 7, s16 & 127])

    pltpu.sync_copy(er_hbm, tab_v)
    for u in range(NSUP):
      r0a, off = sup_base(u)
      pltpu.sync_copy(d2d_hbm.at[pl.ds(r0a, SROWS)], dst_sup)

      @plsc.parallel_loop(0, SUP * CH // 16, 1, unroll=2)
      def p1b(i):
        esl = pl.ds(u * SUP * CH + i * 16, 16)
        d16 = dst_sup[off + i // 5, pl.ds((i % 5) * 16, 16)]
        e = ee_v[esl] + plsc.load_gather(tab_v, [d16 >> 7, d16 & 127])
        e = jnp.where(e >= 0.0, e, e * jnp.float32(0.2))
        ee = jnp.exp(e)
        ee_v[esl] = ee
        plsc.addupdate_scatter(den_v, [d16 >> 7, d16 & 127], ee)

    plsc.subcore_barrier()
    pltpu.sync_copy(den_v, den_sh.at[ident_v], add=True)
    plsc.subcore_barrier()
    pltpu.sync_copy(den_sh, den_v)

    for u in range(NSUP):
      r0a, off = sup_base(u)
      pltpu.sync_copy(d2d_hbm.at[pl.ds(r0a, SROWS)], dst_sup)

      @plsc.parallel_loop(0, SUP * CH // 16, 1, unroll=4)
      def p25(i):
        esl = pl.ds(u * SUP * CH + i * 16, 16)
        d16 = dst_sup[off + i // 5, pl.ds((i % 5) * 16, 16)]
        den16 = plsc.load_gather(den_v, [d16 >> 7, d16 & 127])
        ee_v[esl] = ee_v[esl] / den16

    rbufs = (rows_v, den_v, tab_v)
    semgs = (semg0, semg1, semg2)
    semss = (sems0, sems1, sems2)
    gis = (gi0, gi1, gi2)

    def build_gidx(row, b):
      def mk(j, _):
        o16 = pl.ds(j * 16, 16)
        gis[b][o16] = src_sup[row, o16] * 2 + c
        return 0
      lax.fori_loop(0, CH // 16, mk, 0)

    def gather_start(b):
      pltpu.async_copy(h2_hbm.at[gis[b]], rbufs[b], semgs[b])

    def gather_wait(b):
      pltpu.make_async_copy(h2_hbm.at[gis[b]], rbufs[b], semgs[b]).wait()

    def scatter_start(row, b):
      pltpu.async_copy(rbufs[b], accum.at[dst_sup.at[row]], semss[b],
                       add=True)

    def scatter_wait(b):
      pltpu.make_async_copy(rbufs[b], accum.at[dst_sup.at[0]],
                            semss[b]).wait()

    def scale(eb, b):
      rbuf = rbufs[b]

      @plsc.parallel_loop(0, CH, 1, unroll=8)
      def _(e):
        av = plsc.load_gather(ee_v, [jnp.zeros((16,), jnp.int32) + (eb + e)])
        for w in range(8):
          sl = pl.ds(w * 16, 16)
          rbuf[e, sl] = rbuf[e, sl] * av

    def super_body(u, _):
      r0a, off = sup_base(u)
      pltpu.sync_copy(s2d_hbm.at[pl.ds(r0a, SROWS)], src_sup)
      pltpu.sync_copy(d2d_hbm.at[pl.ds(r0a, SROWS)], dst_sup)
      ebu = u * SUP * CH

      build_gidx(off, 0)
      gather_start(0)

      def tri(i, _):
        for kk in range(3):
          lc = i * 3 + kk
          nb = (kk + 1) % 3

          @pl.when(lc >= 2)
          def _():
            scatter_wait(nb)
          build_gidx(off + lc + 1, nb)
          gather_start(nb)
          gather_wait(kk)
          scale(ebu + lc * CH, kk)
          scatter_start(off + lc, kk)
        return 0
      lax.fori_loop(0, (SUP - 1) // 3, tri, 0)

      gather_wait(0)
      scale(ebu + (SUP - 1) * CH, 0)
      scatter_start(off + SUP - 1, 0)
      scatter_wait(1)
      scatter_wait(2)
      scatter_wait(0)
      return 0
    lax.fori_loop(0, NSUP, super_body, 0)

    plsc.subcore_barrier()
    pltpu.sync_copy(accum.at[pl.ds(abase, RSPAN)],
                    out_hbm.at[c, pl.ds(abase, RSPAN)])

  return k(h2, el2d, er2d, src2d, dst2d)


def _pad_tab(v):
  return v.reshape(TR, 128)


_RB = 400


def _proj_body(x_ref, w_ref, al_ref, ar_ref, h_ref, el_ref, er_ref):
  h = jnp.dot(x_ref[...], w_ref[...], preferred_element_type=jnp.float32)
  h_ref[...] = h
  el_ref[...] = jnp.sum(h * al_ref[...], axis=1, keepdims=True)
  er_ref[...] = jnp.sum(h * ar_ref[...], axis=1, keepdims=True)


def _tc_proj(x, W, al, ar):
  return pl.pallas_call(
      _proj_body,
      grid=(N // _RB,),
      in_specs=[
          pl.BlockSpec((_RB, D), lambda i: (i, 0)),
          pl.BlockSpec((D, D), lambda i: (0, 0)),
          pl.BlockSpec((1, D), lambda i: (0, 0)),
          pl.BlockSpec((1, D), lambda i: (0, 0)),
      ],
      out_specs=[
          pl.BlockSpec((_RB, D), lambda i: (i, 0)),
          pl.BlockSpec((_RB, 1), lambda i: (i, 0)),
          pl.BlockSpec((_RB, 1), lambda i: (i, 0)),
      ],
      out_shape=[
          jax.ShapeDtypeStruct((N, D), jnp.float32),
          jax.ShapeDtypeStruct((NPAD, 1), jnp.float32),
          jax.ShapeDtypeStruct((NPAD, 1), jnp.float32),
      ],
  )(x, W, al.reshape(1, D), ar.reshape(1, D))


def _elu(v):
  return jnp.where(v > 0.0, v, jnp.exp(jnp.minimum(v, 0.0)) - 1.0)


def _mid_body(rst_ref, x_ref, b_ref, w_ref, al_ref, ar_ref,
              y_ref, h_ref, el_ref, er_ref):
  r = jnp.concatenate([rst_ref[0], rst_ref[1]], axis=1)
  y = _elu(r + x_ref[...] + b_ref[...])
  y_ref[...] = y
  h = jnp.dot(y, w_ref[...], preferred_element_type=jnp.float32)
  h_ref[...] = h
  el_ref[...] = jnp.sum(h * al_ref[...], axis=1, keepdims=True)
  er_ref[...] = jnp.sum(h * ar_ref[...], axis=1, keepdims=True)


def _tc_mid(rst, x, b, W, al, ar):
  return pl.pallas_call(
      _mid_body,
      grid=(N // _RB,),
      in_specs=[
          pl.BlockSpec((NC, _RB, 128), lambda i: (0, i, 0)),
          pl.BlockSpec((_RB, D), lambda i: (i, 0)),
          pl.BlockSpec((1, D), lambda i: (0, 0)),
          pl.BlockSpec((D, D), lambda i: (0, 0)),
          pl.BlockSpec((1, D), lambda i: (0, 0)),
          pl.BlockSpec((1, D), lambda i: (0, 0)),
      ],
      out_specs=[
          pl.BlockSpec((_RB, D), lambda i: (i, 0)),
          pl.BlockSpec((_RB, D), lambda i: (i, 0)),
          pl.BlockSpec((_RB, 1), lambda i: (i, 0)),
          pl.BlockSpec((_RB, 1), lambda i: (i, 0)),
      ],
      out_shape=[
          jax.ShapeDtypeStruct((N, D), jnp.float32),
          jax.ShapeDtypeStruct((N, D), jnp.float32),
          jax.ShapeDtypeStruct((NPAD, 1), jnp.float32),
          jax.ShapeDtypeStruct((NPAD, 1), jnp.float32),
      ],
  )(rst, x, b.reshape(1, D), W, al.reshape(1, D), ar.reshape(1, D))


def _fin_body(rst_ref, y_ref, b_ref, o_ref):
  r = jnp.concatenate([rst_ref[0], rst_ref[1]], axis=1)
  o_ref[...] = _elu(r + y_ref[...] + b_ref[...])


def _tc_fin(rst, y, b):
  return pl.pallas_call(
      _fin_body,
      grid=(N // _RB,),
      in_specs=[
          pl.BlockSpec((NC, _RB, 128), lambda i: (0, i, 0)),
          pl.BlockSpec((_RB, D), lambda i: (i, 0)),
          pl.BlockSpec((1, D), lambda i: (0, 0)),
      ],
      out_specs=pl.BlockSpec((_RB, D), lambda i: (i, 0)),
      out_shape=jax.ShapeDtypeStruct((N, D), jnp.float32),
  )(rst, y, b.reshape(1, D))


@jax.jit
def _run(x, src2d, dst2d, W0, al0, ar0, b0, W1, al1, ar1, b1):
  h1, el1, er1 = _tc_proj(x, W0, al0, ar0)
  rst1 = _sc_edge_layer(h1.reshape(2 * N, 128), _pad_tab(el1),
                        _pad_tab(er1), src2d, dst2d)
  y1, h2, el2, er2 = _tc_mid(rst1, x, b0, W1, al1, ar1)
  rst2 = _sc_edge_layer(h2.reshape(2 * N, 128), _pad_tab(el2),
                        _pad_tab(er2), src2d, dst2d)
  return _tc_fin(rst2, y1, b1)


def kernel(x, edge_index, W0, al0, ar0, b0, W1, al1, ar1, b1):
  src2d = edge_index[0].astype(jnp.int32).reshape(E // CH, CH)
  dst2d = edge_index[1].astype(jnp.int32).reshape(E // CH, CH)
  return _run(x, src2d, dst2d, W0, al0, ar0, b0, W1, al1, ar1, b1)


# speedup vs baseline: 1.0029x; 1.0029x over previous
import functools

import jax
import jax.numpy as jnp
from jax import lax
from jax.experimental import pallas as pl
from jax.experimental.pallas import tpu as pltpu
from jax.experimental.pallas import tpu_sc as plsc

N = 10000
E = 160000
D = 256

NC = 2
NS = 16
EPS = E // NS
NPAD = 10240
TR = NPAD // 128
RSPAN = 632
CH = 80
NCH = EPS // CH
SUP = 25
NSUP = NCH // SUP
SROWS = 32


def _sc_edge_layer(h2, el2d, er2d, src2d, dst2d):
  mesh = plsc.VectorSubcoreMesh(core_axis_name="c", subcore_axis_name="s")

  @functools.partial(
      pl.kernel,
      out_type=(jax.ShapeDtypeStruct((NC, N, 128), jnp.float32),
                jax.ShapeDtypeStruct((TR, 128), jnp.float32)),
      mesh=mesh,
      compiler_params=pltpu.CompilerParams(needs_layout_passes=False),
      scratch_types=[
          pltpu.VMEM((TR, 128), jnp.float32),
          pltpu.VMEM((TR, 128), jnp.float32),
          pltpu.VMEM((EPS,), jnp.float32),
          pltpu.VMEM((CH, 128), jnp.float32),
          pltpu.VMEM((SROWS, CH), jnp.int32),
          pltpu.VMEM((SROWS, CH), jnp.int32),
          pltpu.VMEM((CH,), jnp.int32),
          pltpu.VMEM((CH,), jnp.int32),
          pltpu.VMEM((CH,), jnp.int32),
          pltpu.VMEM((TR,), jnp.int32),
          pltpu.SemaphoreType.DMA,
          pltpu.SemaphoreType.DMA,
          pltpu.SemaphoreType.DMA,
          pltpu.SemaphoreType.DMA,
          pltpu.SemaphoreType.DMA,
          pltpu.SemaphoreType.DMA,
          pltpu.VMEM_SHARED((N, 128), jnp.float32),
          pltpu.VMEM_SHARED((TR, 128), jnp.float32),
      ],
  )
  def k(h2_hbm, el_hbm, er_hbm, s2d_hbm, d2d_hbm, out_hbm, den_hbm,
        tab_v, den_v, ee_v, rows_v, src_sup, dst_sup,
        gi0, gi1, gi2, ident_v,
        semg0, semg1, semg2, sems0, sems1, sems2, accum, den_sh):
    c = lax.axis_index("c")
    s = lax.axis_index("s")
    z16 = jnp.zeros((16,), jnp.float32)
    iota16 = lax.iota(jnp.int32, 16)

    def zrows(i, _):
      rows_v[i // 8, pl.ds((i % 8) * 16, 16)] = z16
      return 0
    lax.fori_loop(0, CH * 8, zrows, 0)

    def zden(i, _):
      den_v[i // 8, pl.ds((i % 8) * 16, 16)] = z16
      return 0
    lax.fori_loop(0, TR * 8, zden, 0)

    def mkid(j, _):
      ident_v[pl.ds(j * 16, 16)] = j * 16 + iota16
      return 0
    lax.fori_loop(0, TR // 16, mkid, 0)

    abase = pl.multiple_of((s * (N // NS)) // 8 * 8, 8)
    nfull = RSPAN // CH
    rem = RSPAN - nfull * CH
    for r in range(nfull):
      pltpu.async_copy(rows_v, accum.at[pl.ds(abase + r * CH, CH)], semg0)
    pltpu.async_copy(rows_v.at[pl.ds(0, rem)],
                     accum.at[pl.ds(abase + nfull * CH, rem)], semg1)

    @pl.when(s < TR // 8)
    def _():
      pltpu.async_copy(rows_v.at[pl.ds(0, 8)],
                       den_sh.at[pl.ds(pl.multiple_of(s * 8, 8), 8)], semg2)

    for r in range(nfull):
      pltpu.make_async_copy(
          rows_v, accum.at[pl.ds(abase, CH)], semg0).wait()
    pltpu.make_async_copy(
        rows_v.at[pl.ds(0, rem)], accum.at[pl.ds(abase, rem)], semg1).wait()

    @pl.when(s < TR // 8)
    def _():
      pltpu.make_async_copy(
          rows_v.at[pl.ds(0, 8)], den_sh.at[pl.ds(0, 8)], semg2).wait()

    def sup_base(u):
      r0 = s * NCH + u * SUP
      r0a = pl.multiple_of(r0 // 8 * 8, 8)
      return r0a, r0 - r0a

    pltpu.sync_copy(el_hbm, tab_v)
    for u in range(NSUP):
      r0a, off = sup_base(u)
      pltpu.sync_copy(s2d_hbm.at[pl.ds(r0a, SROWS)], src_sup)

      @plsc.parallel_loop(0, SUP * CH // 16, 1, unroll=4)
      def p1a(i):
        s16 = src_sup[off + i // 5, pl.ds((i % 5) * 16, 16)]
        ee_v[pl.ds(u * SUP * CH + i * 16, 16)] = plsc.load_gather(
            tab_v, [s16 >> 7, s16 & 127])

    pltpu.sync_copy(er_hbm, tab_v)
    for u in range(NSUP):
      r0a, off = sup_base(u)
      pltpu.sync_copy(d2d_hbm.at[pl.ds(r0a, SROWS)], dst_sup)

      @plsc.parallel_loop(0, SUP * CH // 16, 1, unroll=2)
      def p1b(i):
        esl = pl.ds(u * SUP * CH + i * 16, 16)
        d16 = dst_sup[off + i // 5, pl.ds((i % 5) * 16, 16)]
        e = ee_v[esl] + plsc.load_gather(tab_v, [d16 >> 7, d16 & 127])
        e = jnp.where(e >= 0.0, e, e * jnp.float32(0.2))
        ee = jnp.exp(e)
        ee_v[esl] = ee
        plsc.addupdate_scatter(den_v, [d16 >> 7, d16 & 127], ee)

    plsc.subcore_barrier()
    pltpu.sync_copy(den_v, den_sh.at[ident_v], add=True)
    plsc.subcore_barrier()

    @pl.when((s < TR // 8) & (c == 0))
    def _():
      sb = pl.multiple_of(s * 8, 8)
      pltpu.sync_copy(den_sh.at[pl.ds(sb, 8)], den_hbm.at[pl.ds(sb, 8)])

    rbufs = (rows_v, den_v, tab_v)
    semgs = (semg0, semg1, semg2)
    semss = (sems0, sems1, sems2)
    gis = (gi0, gi1, gi2)

    def build_gidx(row, b):
      def mk(j, _):
        o16 = pl.ds(j * 16, 16)
        gis[b][o16] = src_sup[row, o16] * 2 + c
        return 0
      lax.fori_loop(0, CH // 16, mk, 0)

    def gather_start(b):
      pltpu.async_copy(h2_hbm.at[gis[b]], rbufs[b], semgs[b])

    def gather_wait(b):
      pltpu.make_async_copy(h2_hbm.at[gis[b]], rbufs[b], semgs[b]).wait()

    def scatter_start(row, b):
      pltpu.async_copy(rbufs[b], accum.at[dst_sup.at[row]], semss[b],
                       add=True)

    def scatter_wait(b):
      pltpu.make_async_copy(rbufs[b], accum.at[dst_sup.at[0]],
                            semss[b]).wait()

    def scale(eb, b):
      rbuf = rbufs[b]

      @plsc.parallel_loop(0, CH, 1, unroll=4)
      def _(e):
        av = plsc.load_gather(ee_v, [jnp.zeros((16,), jnp.int32) + (eb + e)])
        for w in range(8):
          sl = pl.ds(w * 16, 16)
          rbuf[e, sl] = rbuf[e, sl] * av

    def super_body(u, _):
      r0a, off = sup_base(u)
      pltpu.sync_copy(s2d_hbm.at[pl.ds(r0a, SROWS)], src_sup)
      pltpu.sync_copy(d2d_hbm.at[pl.ds(r0a, SROWS)], dst_sup)
      ebu = u * SUP * CH

      build_gidx(off, 0)
      gather_start(0)

      def tri(i, _):
        for kk in range(3):
          lc = i * 3 + kk
          nb = (kk + 1) % 3

          @pl.when(lc >= 2)
          def _():
            scatter_wait(nb)
          build_gidx(off + lc + 1, nb)
          gather_start(nb)
          gather_wait(kk)
          scale(ebu + lc * CH, kk)
          scatter_start(off + lc, kk)
        return 0
      lax.fori_loop(0, (SUP - 1) // 3, tri, 0)

      gather_wait(0)
      scale(ebu + (SUP - 1) * CH, 0)
      scatter_start(off + SUP - 1, 0)
      scatter_wait(1)
      scatter_wait(2)
      scatter_wait(0)
      return 0
    lax.fori_loop(0, NSUP, super_body, 0)

    plsc.subcore_barrier()
    pltpu.sync_copy(accum.at[pl.ds(abase, RSPAN)],
                    out_hbm.at[c, pl.ds(abase, RSPAN)])

  return k(h2, el2d, er2d, src2d, dst2d)


def _pad_tab(v):
  return v.reshape(TR, 128)


_RB = 400


def _proj_body(x_ref, w_ref, al_ref, ar_ref, h_ref, el_ref, er_ref):
  h = jnp.dot(x_ref[...], w_ref[...], preferred_element_type=jnp.float32)
  h_ref[...] = h
  el_ref[...] = jnp.sum(h * al_ref[...], axis=1, keepdims=True)
  er_ref[...] = jnp.sum(h * ar_ref[...], axis=1, keepdims=True)


def _tc_proj(x, W, al, ar):
  return pl.pallas_call(
      _proj_body,
      grid=(N // _RB,),
      in_specs=[
          pl.BlockSpec((_RB, D), lambda i: (i, 0)),
          pl.BlockSpec((D, D), lambda i: (0, 0)),
          pl.BlockSpec((1, D), lambda i: (0, 0)),
          pl.BlockSpec((1, D), lambda i: (0, 0)),
      ],
      out_specs=[
          pl.BlockSpec((_RB, D), lambda i: (i, 0)),
          pl.BlockSpec((_RB, 1), lambda i: (i, 0)),
          pl.BlockSpec((_RB, 1), lambda i: (i, 0)),
      ],
      out_shape=[
          jax.ShapeDtypeStruct((N, D), jnp.float32),
          jax.ShapeDtypeStruct((NPAD, 1), jnp.float32),
          jax.ShapeDtypeStruct((NPAD, 1), jnp.float32),
      ],
  )(x, W, al.reshape(1, D), ar.reshape(1, D))


def _elu(v):
  return jnp.where(v > 0.0, v, jnp.exp(jnp.minimum(v, 0.0)) - 1.0)


def _mid_body(rst_ref, den_ref, x_ref, b_ref, w_ref, al_ref, ar_ref,
              y_ref, h_ref, el_ref, er_ref):
  dr = den_ref[...]
  inv = jnp.where(dr > 0.0, 1.0 / dr, 0.0)
  r = jnp.concatenate([rst_ref[0], rst_ref[1]], axis=1) * inv
  y = _elu(r + x_ref[...] + b_ref[...])
  y_ref[...] = y
  h = jnp.dot(y, w_ref[...], preferred_element_type=jnp.float32)
  h_ref[...] = h
  el_ref[...] = jnp.sum(h * al_ref[...], axis=1, keepdims=True)
  er_ref[...] = jnp.sum(h * ar_ref[...], axis=1, keepdims=True)


def _tc_mid(rst, den, x, b, W, al, ar):
  return pl.pallas_call(
      _mid_body,
      grid=(N // _RB,),
      in_specs=[
          pl.BlockSpec((NC, _RB, 128), lambda i: (0, i, 0)),
          pl.BlockSpec((_RB, 1), lambda i: (i, 0)),
          pl.BlockSpec((_RB, D), lambda i: (i, 0)),
          pl.BlockSpec((1, D), lambda i: (0, 0)),
          pl.BlockSpec((D, D), lambda i: (0, 0)),
          pl.BlockSpec((1, D), lambda i: (0, 0)),
          pl.BlockSpec((1, D), lambda i: (0, 0)),
      ],
      out_specs=[
          pl.BlockSpec((_RB, D), lambda i: (i, 0)),
          pl.BlockSpec((_RB, D), lambda i: (i, 0)),
          pl.BlockSpec((_RB, 1), lambda i: (i, 0)),
          pl.BlockSpec((_RB, 1), lambda i: (i, 0)),
      ],
      out_shape=[
          jax.ShapeDtypeStruct((N, D), jnp.float32),
          jax.ShapeDtypeStruct((N, D), jnp.float32),
          jax.ShapeDtypeStruct((NPAD, 1), jnp.float32),
          jax.ShapeDtypeStruct((NPAD, 1), jnp.float32),
      ],
  )(rst, den.reshape(NPAD, 1), x, b.reshape(1, D), W,
    al.reshape(1, D), ar.reshape(1, D))


def _fin_body(rst_ref, den_ref, y_ref, b_ref, o_ref):
  dr = den_ref[...]
  inv = jnp.where(dr > 0.0, 1.0 / dr, 0.0)
  r = jnp.concatenate([rst_ref[0], rst_ref[1]], axis=1) * inv
  o_ref[...] = _elu(r + y_ref[...] + b_ref[...])


def _tc_fin(rst, den, y, b):
  return pl.pallas_call(
      _fin_body,
      grid=(N // _RB,),
      in_specs=[
          pl.BlockSpec((NC, _RB, 128), lambda i: (0, i, 0)),
          pl.BlockSpec((_RB, 1), lambda i: (i, 0)),
          pl.BlockSpec((_RB, D), lambda i: (i, 0)),
          pl.BlockSpec((1, D), lambda i: (0, 0)),
      ],
      out_specs=pl.BlockSpec((_RB, D), lambda i: (i, 0)),
      out_shape=jax.ShapeDtypeStruct((N, D), jnp.float32),
  )(rst, den.reshape(NPAD, 1), y, b.reshape(1, D))


@jax.jit
def _run(x, src2d, dst2d, W0, al0, ar0, b0, W1, al1, ar1, b1):
  h1, el1, er1 = _tc_proj(x, W0, al0, ar0)
  rst1, den1 = _sc_edge_layer(h1.reshape(2 * N, 128), _pad_tab(el1),
                              _pad_tab(er1), src2d, dst2d)
  y1, h2, el2, er2 = _tc_mid(rst1, den1, x, b0, W1, al1, ar1)
  rst2, den2 = _sc_edge_layer(h2.reshape(2 * N, 128), _pad_tab(el2),
                              _pad_tab(er2), src2d, dst2d)
  return _tc_fin(rst2, den2, y1, b1)


def kernel(x, edge_index, W0, al0, ar0, b0, W1, al1, ar1, b1):
  src2d = edge_index[0].astype(jnp.int32).reshape(E // CH, CH)
  dst2d = edge_index[1].astype(jnp.int32).reshape(E // CH, CH)
  return _run(x, src2d, dst2d, W0, al0, ar0, b0, W1, al1, ar1, b1)


# speedup vs baseline: 1.0226x; 1.0197x over previous
import functools

import jax
import jax.numpy as jnp
from jax import lax
from jax.experimental import pallas as pl
from jax.experimental.pallas import tpu as pltpu
from jax.experimental.pallas import tpu_sc as plsc

N = 10000
E = 160000
D = 256

NC = 2
NS = 16
EPS = E // NS
NPAD = 10240
TR = NPAD // 128
RSPAN = 632
CH = 80
NCH = EPS // CH
SUP = 25
NSUP = NCH // SUP
SROWS = 32


def _sc_edge_layer(h2, el2d, er2d, src2d, dst2d):
  mesh = plsc.VectorSubcoreMesh(core_axis_name="c", subcore_axis_name="s")

  @functools.partial(
      pl.kernel,
      out_type=(jax.ShapeDtypeStruct((NC, N, 128), jnp.float32),
                jax.ShapeDtypeStruct((TR, 128), jnp.float32)),
      mesh=mesh,
      compiler_params=pltpu.CompilerParams(needs_layout_passes=False),
      scratch_types=[
          pltpu.VMEM((TR, 128), jnp.float32),
          pltpu.VMEM((TR, 128), jnp.float32),
          pltpu.VMEM((EPS,), jnp.float32),
          pltpu.VMEM((CH, 128), jnp.float32),
          pltpu.VMEM((SROWS, CH), jnp.int32),
          pltpu.VMEM((SROWS, CH), jnp.int32),
          pltpu.VMEM((CH,), jnp.int32),
          pltpu.VMEM((CH,), jnp.int32),
          pltpu.VMEM((CH,), jnp.int32),
          pltpu.VMEM((CH,), jnp.int32),
          pltpu.VMEM((CH,), jnp.int32),
          pltpu.VMEM((CH,), jnp.int32),
          pltpu.VMEM((TR,), jnp.int32),
          pltpu.SemaphoreType.DMA,
          pltpu.SemaphoreType.DMA,
          pltpu.SemaphoreType.DMA,
          pltpu.SemaphoreType.DMA,
          pltpu.SemaphoreType.DMA,
          pltpu.SemaphoreType.DMA,
          pltpu.VMEM_SHARED((N, 128), jnp.float32),
          pltpu.VMEM_SHARED((TR, 128), jnp.float32),
      ],
  )
  def k(h2_hbm, el_hbm, er_hbm, s2d_hbm, d2d_hbm, out_hbm, den_hbm,
        tab_v, den_v, ee_v, rows_v, src_sup, dst_sup,
        gi0, gi1, gi2, si0, si1, si2, ident_v,
        semg0, semg1, semg2, sems0, sems1, sems2, accum, den_sh):
    c = lax.axis_index("c")
    s = lax.axis_index("s")
    z16 = jnp.zeros((16,), jnp.float32)
    iota16 = lax.iota(jnp.int32, 16)

    def zrows(i, _):
      rows_v[i // 8, pl.ds((i % 8) * 16, 16)] = z16
      return 0
    lax.fori_loop(0, CH * 8, zrows, 0)

    def zden(i, _):
      den_v[i // 8, pl.ds((i % 8) * 16, 16)] = z16
      return 0
    lax.fori_loop(0, TR * 8, zden, 0)

    def mkid(j, _):
      ident_v[pl.ds(j * 16, 16)] = j * 16 + iota16
      return 0
    lax.fori_loop(0, TR // 16, mkid, 0)

    abase = pl.multiple_of((s * (N // NS)) // 8 * 8, 8)
    nfull = RSPAN // CH
    rem = RSPAN - nfull * CH
    for r in range(nfull):
      pltpu.async_copy(rows_v, accum.at[pl.ds(abase + r * CH, CH)], semg0)
    pltpu.async_copy(rows_v.at[pl.ds(0, rem)],
                     accum.at[pl.ds(abase + nfull * CH, rem)], semg1)

    @pl.when(s < TR // 8)
    def _():
      pltpu.async_copy(rows_v.at[pl.ds(0, 8)],
                       den_sh.at[pl.ds(pl.multiple_of(s * 8, 8), 8)], semg2)

    for r in range(nfull):
      pltpu.make_async_copy(
          rows_v, accum.at[pl.ds(abase, CH)], semg0).wait()
    pltpu.make_async_copy(
        rows_v.at[pl.ds(0, rem)], accum.at[pl.ds(abase, rem)], semg1).wait()

    @pl.when(s < TR // 8)
    def _():
      pltpu.make_async_copy(
          rows_v.at[pl.ds(0, 8)], den_sh.at[pl.ds(0, 8)], semg2).wait()

    def sup_base(u):
      r0 = s * NCH + u * SUP
      r0a = pl.multiple_of(r0 // 8 * 8, 8)
      return r0a, r0 - r0a

    pltpu.sync_copy(el_hbm, tab_v)
    for u in range(NSUP):
      r0a, off = sup_base(u)
      pltpu.sync_copy(s2d_hbm.at[pl.ds(r0a, SROWS)], src_sup)

      @plsc.parallel_loop(0, SUP * CH // 16, 1, unroll=4)
      def p1a(i):
        s16 = src_sup[off + i // 5, pl.ds((i % 5) * 16, 16)]
        ee_v[pl.ds(u * SUP * CH + i * 16, 16)] = plsc.load_gather(
            tab_v, [s16 >> 7, s16 & 127])

    pltpu.sync_copy(er_hbm, tab_v)
    for u in range(NSUP):
      r0a, off = sup_base(u)
      pltpu.sync_copy(d2d_hbm.at[pl.ds(r0a, SROWS)], dst_sup)

      @plsc.parallel_loop(0, SUP * CH // 16, 1, unroll=2)
      def p1b(i):
        esl = pl.ds(u * SUP * CH + i * 16, 16)
        d16 = dst_sup[off + i // 5, pl.ds((i % 5) * 16, 16)]
        e = ee_v[esl] + plsc.load_gather(tab_v, [d16 >> 7, d16 & 127])
        e = jnp.where(e >= 0.0, e, e * jnp.float32(0.2))
        ee = jnp.exp(e)
        ee_v[esl] = ee
        plsc.addupdate_scatter(den_v, [d16 >> 7, d16 & 127], ee)

    plsc.subcore_barrier()
    pltpu.sync_copy(den_v, den_sh.at[ident_v], add=True)
    plsc.subcore_barrier()

    @pl.when((s < TR // 8) & (c == 0))
    def _():
      sb = pl.multiple_of(s * 8, 8)
      pltpu.sync_copy(den_sh.at[pl.ds(sb, 8)], den_hbm.at[pl.ds(sb, 8)])

    rbufs = (rows_v, den_v, tab_v)
    semgs = (semg0, semg1, semg2)
    semss = (sems0, sems1, sems2)
    gis = (gi0, gi1, gi2)
    sis = (si0, si1, si2)

    def prep(m1, b):
      u = m1 // SUP
      r0a = pl.multiple_of((s * NCH + u * SUP) // 8 * 8, 8)

      @pl.when(m1 % SUP == 0)
      def _():
        pltpu.sync_copy(s2d_hbm.at[pl.ds(r0a, SROWS)], src_sup)
        pltpu.sync_copy(d2d_hbm.at[pl.ds(r0a, SROWS)], dst_sup)

      row = s * NCH + m1 - r0a

      def mk(j, _):
        o16 = pl.ds(j * 16, 16)
        gis[b][o16] = src_sup[row, o16] * 2 + c
        sis[b][o16] = dst_sup[row, o16]
        return 0
      lax.fori_loop(0, CH // 16, mk, 0)

    def gather_start(b):
      pltpu.async_copy(h2_hbm.at[gis[b]], rbufs[b], semgs[b])

    def gather_wait(b):
      pltpu.make_async_copy(h2_hbm.at[gis[b]], rbufs[b], semgs[b]).wait()

    def scatter_start(b):
      pltpu.async_copy(rbufs[b], accum.at[sis[b]], semss[b], add=True)

    def scatter_wait(b):
      pltpu.make_async_copy(rbufs[b], accum.at[sis[b]], semss[b]).wait()

    def scale(eb, b):
      rbuf = rbufs[b]

      @plsc.parallel_loop(0, CH, 1, unroll=4)
      def _(e):
        av = plsc.load_gather(ee_v, [jnp.zeros((16,), jnp.int32) + (eb + e)])
        for w in range(8):
          sl = pl.ds(w * 16, 16)
          rbuf[e, sl] = rbuf[e, sl] * av

    prep(0, 0)
    gather_start(0)

    def tri(i, _):
      for kk in range(3):
        m = i * 3 + kk
        nb = (kk + 1) % 3

        @pl.when(m >= 2)
        def _():
          scatter_wait(nb)
        prep(m + 1, nb)
        gather_start(nb)
        gather_wait(kk)
        scale(m * CH, kk)
        scatter_start(kk)
      return 0
    lax.fori_loop(0, (NCH - 2) // 3, tri, 0)

    scatter_wait(1)
    prep(NCH - 1, 1)
    gather_start(1)
    gather_wait(0)
    scale((NCH - 2) * CH, 0)
    scatter_start(0)
    scatter_wait(2)
    gather_wait(1)
    scale((NCH - 1) * CH, 1)
    scatter_start(1)
    scatter_wait(0)
    scatter_wait(1)

    plsc.subcore_barrier()
    pltpu.sync_copy(accum.at[pl.ds(abase, RSPAN)],
                    out_hbm.at[c, pl.ds(abase, RSPAN)])

  return k(h2, el2d, er2d, src2d, dst2d)


def _pad_tab(v):
  return v.reshape(TR, 128)


_RB = 400


def _proj_body(x_ref, w_ref, al_ref, ar_ref, h_ref, el_ref, er_ref):
  h = jnp.dot(x_ref[...], w_ref[...], preferred_element_type=jnp.float32)
  h_ref[...] = h
  el_ref[...] = jnp.sum(h * al_ref[...], axis=1, keepdims=True)
  er_ref[...] = jnp.sum(h * ar_ref[...], axis=1, keepdims=True)


def _tc_proj(x, W, al, ar):
  return pl.pallas_call(
      _proj_body,
      grid=(N // _RB,),
      in_specs=[
          pl.BlockSpec((_RB, D), lambda i: (i, 0)),
          pl.BlockSpec((D, D), lambda i: (0, 0)),
          pl.BlockSpec((1, D), lambda i: (0, 0)),
          pl.BlockSpec((1, D), lambda i: (0, 0)),
      ],
      out_specs=[
          pl.BlockSpec((_RB, D), lambda i: (i, 0)),
          pl.BlockSpec((_RB, 1), lambda i: (i, 0)),
          pl.BlockSpec((_RB, 1), lambda i: (i, 0)),
      ],
      out_shape=[
          jax.ShapeDtypeStruct((N, D), jnp.float32),
          jax.ShapeDtypeStruct((NPAD, 1), jnp.float32),
          jax.ShapeDtypeStruct((NPAD, 1), jnp.float32),
      ],
  )(x, W, al.reshape(1, D), ar.reshape(1, D))


def _elu(v):
  return jnp.where(v > 0.0, v, jnp.exp(jnp.minimum(v, 0.0)) - 1.0)


def _mid_body(rst_ref, den_ref, x_ref, b_ref, w_ref, al_ref, ar_ref,
              y_ref, h_ref, el_ref, er_ref):
  dr = den_ref[...]
  inv = jnp.where(dr > 0.0, 1.0 / dr, 0.0)
  r = jnp.concatenate([rst_ref[0], rst_ref[1]], axis=1) * inv
  y = _elu(r + x_ref[...] + b_ref[...])
  y_ref[...] = y
  h = jnp.dot(y, w_ref[...], preferred_element_type=jnp.float32)
  h_ref[...] = h
  el_ref[...] = jnp.sum(h * al_ref[...], axis=1, keepdims=True)
  er_ref[...] = jnp.sum(h * ar_ref[...], axis=1, keepdims=True)


def _tc_mid(rst, den, x, b, W, al, ar):
  return pl.pallas_call(
      _mid_body,
      grid=(N // _RB,),
      in_specs=[
          pl.BlockSpec((NC, _RB, 128), lambda i: (0, i, 0)),
          pl.BlockSpec((_RB, 1), lambda i: (i, 0)),
          pl.BlockSpec((_RB, D), lambda i: (i, 0)),
          pl.BlockSpec((1, D), lambda i: (0, 0)),
          pl.BlockSpec((D, D), lambda i: (0, 0)),
          pl.BlockSpec((1, D), lambda i: (0, 0)),
          pl.BlockSpec((1, D), lambda i: (0, 0)),
      ],
      out_specs=[
          pl.BlockSpec((_RB, D), lambda i: (i, 0)),
          pl.BlockSpec((_RB, D), lambda i: (i, 0)),
          pl.BlockSpec((_RB, 1), lambda i: (i, 0)),
          pl.BlockSpec((_RB, 1), lambda i: (i, 0)),
      ],
      out_shape=[
          jax.ShapeDtypeStruct((N, D), jnp.float32),
          jax.ShapeDtypeStruct((N, D), jnp.float32),
          jax.ShapeDtypeStruct((NPAD, 1), jnp.float32),
          jax.ShapeDtypeStruct((NPAD, 1), jnp.float32),
      ],
  )(rst, den.reshape(NPAD, 1), x, b.reshape(1, D), W,
    al.reshape(1, D), ar.reshape(1, D))


def _fin_body(rst_ref, den_ref, y_ref, b_ref, o_ref):
  dr = den_ref[...]
  inv = jnp.where(dr > 0.0, 1.0 / dr, 0.0)
  r = jnp.concatenate([rst_ref[0], rst_ref[1]], axis=1) * inv
  o_ref[...] = _elu(r + y_ref[...] + b_ref[...])


def _tc_fin(rst, den, y, b):
  return pl.pallas_call(
      _fin_body,
      grid=(N // _RB,),
      in_specs=[
          pl.BlockSpec((NC, _RB, 128), lambda i: (0, i, 0)),
          pl.BlockSpec((_RB, 1), lambda i: (i, 0)),
          pl.BlockSpec((_RB, D), lambda i: (i, 0)),
          pl.BlockSpec((1, D), lambda i: (0, 0)),
      ],
      out_specs=pl.BlockSpec((_RB, D), lambda i: (i, 0)),
      out_shape=jax.ShapeDtypeStruct((N, D), jnp.float32),
  )(rst, den.reshape(NPAD, 1), y, b.reshape(1, D))


@jax.jit
def _run(x, src2d, dst2d, W0, al0, ar0, b0, W1, al1, ar1, b1):
  h1, el1, er1 = _tc_proj(x, W0, al0, ar0)
  rst1, den1 = _sc_edge_layer(h1.reshape(2 * N, 128), _pad_tab(el1),
                              _pad_tab(er1), src2d, dst2d)
  y1, h2, el2, er2 = _tc_mid(rst1, den1, x, b0, W1, al1, ar1)
  rst2, den2 = _sc_edge_layer(h2.reshape(2 * N, 128), _pad_tab(el2),
                              _pad_tab(er2), src2d, dst2d)
  return _tc_fin(rst2, den2, y1, b1)


def kernel(x, edge_index, W0, al0, ar0, b0, W1, al1, ar1, b1):
  src2d = edge_index[0].astype(jnp.int32).reshape(E // CH, CH)
  dst2d = edge_index[1].astype(jnp.int32).reshape(E // CH, CH)
  return _run(x, src2d, dst2d, W0, al0, ar0, b0, W1, al1, ar1, b1)


# speedup vs baseline: 1.0901x; 1.0660x over previous
import functools

import jax
import jax.numpy as jnp
from jax import lax
from jax.experimental import pallas as pl
from jax.experimental.pallas import tpu as pltpu
from jax.experimental.pallas import tpu_sc as plsc

N = 10000
E = 160000
D = 256

NC = 2
NS = 16
EPS = E // NS
NPAD = 10240
TR = NPAD // 128
RSPAN = 632
CH = 80
NCH = EPS // CH
SUP = 25
NSUP = NCH // SUP
SROWS = 32


def _sc_edge_layer(h2, el2d, er2d, src2d, dst2d):
  mesh = plsc.VectorSubcoreMesh(core_axis_name="c", subcore_axis_name="s")

  @functools.partial(
      pl.kernel,
      out_type=(jax.ShapeDtypeStruct((NC, N, 128), jnp.float32),
                jax.ShapeDtypeStruct((TR, 128), jnp.float32)),
      mesh=mesh,
      compiler_params=pltpu.CompilerParams(needs_layout_passes=False),
      scratch_types=[
          pltpu.VMEM((TR, 128), jnp.float32),
          pltpu.VMEM((TR, 128), jnp.float32),
          pltpu.VMEM((EPS,), jnp.float32),
          pltpu.VMEM((CH, 128), jnp.float32),
          pltpu.VMEM((SROWS, CH), jnp.int32),
          pltpu.VMEM((SROWS, CH), jnp.int32),
          pltpu.VMEM((CH,), jnp.int32),
          pltpu.VMEM((CH,), jnp.int32),
          pltpu.VMEM((CH,), jnp.int32),
          pltpu.VMEM((CH,), jnp.int32),
          pltpu.VMEM((CH,), jnp.int32),
          pltpu.VMEM((CH,), jnp.int32),
          pltpu.VMEM((TR,), jnp.int32),
          pltpu.SemaphoreType.DMA,
          pltpu.SemaphoreType.DMA,
          pltpu.SemaphoreType.DMA,
          pltpu.SemaphoreType.DMA,
          pltpu.SemaphoreType.DMA,
          pltpu.SemaphoreType.DMA,
          pltpu.VMEM_SHARED((N, 128), jnp.float32),
          pltpu.VMEM_SHARED((TR, 128), jnp.float32),
      ],
  )
  def k(h2_hbm, el_hbm, er_hbm, s2d_hbm, d2d_hbm, out_hbm, den_hbm,
        tab_v, den_v, ee_v, rows_v, src_sup, dst_sup,
        gi0, gi1, gi2, si0, si1, si2, ident_v,
        semg0, semg1, semg2, sems0, sems1, sems2, accum, den_sh):
    c = lax.axis_index("c")
    s = lax.axis_index("s")
    z16 = jnp.zeros((16,), jnp.float32)
    iota16 = lax.iota(jnp.int32, 16)

    def zrows(i, _):
      rows_v[i // 8, pl.ds((i % 8) * 16, 16)] = z16
      return 0
    lax.fori_loop(0, CH * 8, zrows, 0)

    def zden(i, _):
      den_v[i // 8, pl.ds((i % 8) * 16, 16)] = z16
      return 0
    lax.fori_loop(0, TR * 8, zden, 0)

    def mkid(j, _):
      ident_v[pl.ds(j * 16, 16)] = j * 16 + iota16
      return 0
    lax.fori_loop(0, TR // 16, mkid, 0)

    abase = pl.multiple_of((s * (N // NS)) // 8 * 8, 8)
    nfull = RSPAN // CH
    rem = RSPAN - nfull * CH
    for r in range(nfull):
      pltpu.async_copy(rows_v, accum.at[pl.ds(abase + r * CH, CH)], semg0)
    pltpu.async_copy(rows_v.at[pl.ds(0, rem)],
                     accum.at[pl.ds(abase + nfull * CH, rem)], semg1)

    @pl.when(s < TR // 8)
    def _():
      pltpu.async_copy(rows_v.at[pl.ds(0, 8)],
                       den_sh.at[pl.ds(pl.multiple_of(s * 8, 8), 8)], semg2)

    for r in range(nfull):
      pltpu.make_async_copy(
          rows_v, accum.at[pl.ds(abase, CH)], semg0).wait()
    pltpu.make_async_copy(
        rows_v.at[pl.ds(0, rem)], accum.at[pl.ds(abase, rem)], semg1).wait()

    @pl.when(s < TR // 8)
    def _():
      pltpu.make_async_copy(
          rows_v.at[pl.ds(0, 8)], den_sh.at[pl.ds(0, 8)], semg2).wait()

    def sup_base(u):
      r0 = s * NCH + u * SUP
      r0a = pl.multiple_of(r0 // 8 * 8, 8)
      return r0a, r0 - r0a

    pltpu.sync_copy(el_hbm, tab_v)
    for u in range(NSUP):
      r0a, off = sup_base(u)
      pltpu.sync_copy(s2d_hbm.at[pl.ds(r0a, SROWS)], src_sup)

      @plsc.parallel_loop(0, SUP * CH // 16, 1, unroll=4)
      def p1a(i):
        s16 = src_sup[off + i // 5, pl.ds((i % 5) * 16, 16)]
        ee_v[pl.ds(u * SUP * CH + i * 16, 16)] = plsc.load_gather(
            tab_v, [s16 >> 7, s16 & 127])

    pltpu.sync_copy(er_hbm, tab_v)
    for u in range(NSUP):
      r0a, off = sup_base(u)
      pltpu.sync_copy(d2d_hbm.at[pl.ds(r0a, SROWS)], dst_sup)

      @plsc.parallel_loop(0, SUP * CH // 16, 1, unroll=2)
      def p1b(i):
        esl = pl.ds(u * SUP * CH + i * 16, 16)
        d16 = dst_sup[off + i // 5, pl.ds((i % 5) * 16, 16)]
        e = ee_v[esl] + plsc.load_gather(tab_v, [d16 >> 7, d16 & 127])
        e = jnp.where(e >= 0.0, e, e * jnp.float32(0.2))
        ee = jnp.exp(e)
        ee_v[esl] = ee
        plsc.addupdate_scatter(den_v, [d16 >> 7, d16 & 127], ee)

    plsc.subcore_barrier()
    pltpu.sync_copy(den_v, den_sh.at[ident_v], add=True)
    plsc.subcore_barrier()

    @pl.when((s < TR // 8) & (c == 0))
    def _():
      sb = pl.multiple_of(s * 8, 8)
      pltpu.sync_copy(den_sh.at[pl.ds(sb, 8)], den_hbm.at[pl.ds(sb, 8)])

    rbufs = (rows_v, den_v, tab_v)
    semgs = (semg0, semg1, semg2)
    semss = (sems0, sems1, sems2)
    gis = (gi0, gi1, gi2)
    sis = (si0, si1, si2)

    def prep(m1, b):
      u = m1 // SUP
      r0a = pl.multiple_of((s * NCH + u * SUP) // 8 * 8, 8)

      @pl.when(m1 % SUP == 0)
      def _():
        pltpu.sync_copy(s2d_hbm.at[pl.ds(r0a, SROWS)], src_sup)
        pltpu.sync_copy(d2d_hbm.at[pl.ds(r0a, SROWS)], dst_sup)

      row = s * NCH + m1 - r0a

      def mk(j, _):
        o16 = pl.ds(j * 16, 16)
        gis[b][o16] = src_sup[row, o16] * 2 + c
        sis[b][o16] = dst_sup[row, o16]
        return 0
      lax.fori_loop(0, CH // 16, mk, 0)

    def gather_start(b):
      pltpu.async_copy(h2_hbm.at[gis[b]], rbufs[b], semgs[b])

    def gather_wait(b):
      pltpu.make_async_copy(h2_hbm.at[gis[b]], rbufs[b], semgs[b]).wait()

    def scatter_start(b):
      pltpu.async_copy(rbufs[b], accum.at[sis[b]], semss[b], add=True)

    def scatter_wait(b):
      pltpu.make_async_copy(rbufs[b], accum.at[sis[b]], semss[b]).wait()

    def scale(eb, b):
      rbuf = rbufs[b]

      @plsc.parallel_loop(0, CH, 1, unroll=4)
      def _(e):
        av = plsc.load_gather(ee_v, [jnp.zeros((16,), jnp.int32) + (eb + e)])
        for w in range(8):
          sl = pl.ds(w * 16, 16)
          rbuf[e, sl] = rbuf[e, sl] * av

    prep(0, 0)
    gather_start(0)

    def tri(i, _):
      for kk in range(3):
        m = i * 3 + kk
        nb = (kk + 1) % 3

        @pl.when(m >= 2)
        def _():
          scatter_wait(nb)
        prep(m + 1, nb)
        gather_start(nb)
        gather_wait(kk)
        scale(m * CH, kk)
        scatter_start(kk)
      return 0
    lax.fori_loop(0, (NCH - 2) // 3, tri, 0)

    scatter_wait(1)
    prep(NCH - 1, 1)
    gather_start(1)
    gather_wait(0)
    scale((NCH - 2) * CH, 0)
    scatter_start(0)
    scatter_wait(2)
    gather_wait(1)
    scale((NCH - 1) * CH, 1)
    scatter_start(1)
    scatter_wait(0)
    scatter_wait(1)

    plsc.subcore_barrier()
    pltpu.sync_copy(accum.at[pl.ds(abase, RSPAN)],
                    out_hbm.at[c, pl.ds(abase, RSPAN)])

  return k(h2, el2d, er2d, src2d, dst2d)


def _pad_tab(v):
  return v.reshape(TR, 128)


_RB = 1000


def _proj_body(x_ref, w_ref, al_ref, ar_ref, h_ref, el_ref, er_ref):
  h = jnp.dot(x_ref[...], w_ref[...], preferred_element_type=jnp.float32)
  h_ref[...] = h
  el_ref[...] = jnp.sum(h * al_ref[...], axis=1, keepdims=True)
  er_ref[...] = jnp.sum(h * ar_ref[...], axis=1, keepdims=True)


def _tc_proj(x, W, al, ar):
  return pl.pallas_call(
      _proj_body,
      grid=(N // _RB,),
      in_specs=[
          pl.BlockSpec((_RB, D), lambda i: (i, 0)),
          pl.BlockSpec((D, D), lambda i: (0, 0)),
          pl.BlockSpec((1, D), lambda i: (0, 0)),
          pl.BlockSpec((1, D), lambda i: (0, 0)),
      ],
      out_specs=[
          pl.BlockSpec((_RB, D), lambda i: (i, 0)),
          pl.BlockSpec((_RB, 1), lambda i: (i, 0)),
          pl.BlockSpec((_RB, 1), lambda i: (i, 0)),
      ],
      out_shape=[
          jax.ShapeDtypeStruct((N, D), jnp.float32),
          jax.ShapeDtypeStruct((NPAD, 1), jnp.float32),
          jax.ShapeDtypeStruct((NPAD, 1), jnp.float32),
      ],
  )(x, W, al.reshape(1, D), ar.reshape(1, D))


def _elu(v):
  return jnp.where(v > 0.0, v, jnp.exp(jnp.minimum(v, 0.0)) - 1.0)


def _mid_body(rst_ref, den_ref, x_ref, b_ref, w_ref, al_ref, ar_ref,
              y_ref, h_ref, el_ref, er_ref):
  dr = den_ref[...]
  inv = jnp.where(dr > 0.0, 1.0 / dr, 0.0)
  r = jnp.concatenate([rst_ref[0], rst_ref[1]], axis=1) * inv
  y = _elu(r + x_ref[...] + b_ref[...])
  y_ref[...] = y
  h = jnp.dot(y, w_ref[...], preferred_element_type=jnp.float32)
  h_ref[...] = h
  el_ref[...] = jnp.sum(h * al_ref[...], axis=1, keepdims=True)
  er_ref[...] = jnp.sum(h * ar_ref[...], axis=1, keepdims=True)


def _tc_mid(rst, den, x, b, W, al, ar):
  return pl.pallas_call(
      _mid_body,
      grid=(N // _RB,),
      in_specs=[
          pl.BlockSpec((NC, _RB, 128), lambda i: (0, i, 0)),
          pl.BlockSpec((_RB, 1), lambda i: (i, 0)),
          pl.BlockSpec((_RB, D), lambda i: (i, 0)),
          pl.BlockSpec((1, D), lambda i: (0, 0)),
          pl.BlockSpec((D, D), lambda i: (0, 0)),
          pl.BlockSpec((1, D), lambda i: (0, 0)),
          pl.BlockSpec((1, D), lambda i: (0, 0)),
      ],
      out_specs=[
          pl.BlockSpec((_RB, D), lambda i: (i, 0)),
          pl.BlockSpec((_RB, D), lambda i: (i, 0)),
          pl.BlockSpec((_RB, 1), lambda i: (i, 0)),
          pl.BlockSpec((_RB, 1), lambda i: (i, 0)),
      ],
      out_shape=[
          jax.ShapeDtypeStruct((N, D), jnp.float32),
          jax.ShapeDtypeStruct((N, D), jnp.float32),
          jax.ShapeDtypeStruct((NPAD, 1), jnp.float32),
          jax.ShapeDtypeStruct((NPAD, 1), jnp.float32),
      ],
  )(rst, den.reshape(NPAD, 1), x, b.reshape(1, D), W,
    al.reshape(1, D), ar.reshape(1, D))


def _fin_body(rst_ref, den_ref, y_ref, b_ref, o_ref):
  dr = den_ref[...]
  inv = jnp.where(dr > 0.0, 1.0 / dr, 0.0)
  r = jnp.concatenate([rst_ref[0], rst_ref[1]], axis=1) * inv
  o_ref[...] = _elu(r + y_ref[...] + b_ref[...])


def _tc_fin(rst, den, y, b):
  return pl.pallas_call(
      _fin_body,
      grid=(N // _RB,),
      in_specs=[
          pl.BlockSpec((NC, _RB, 128), lambda i: (0, i, 0)),
          pl.BlockSpec((_RB, 1), lambda i: (i, 0)),
          pl.BlockSpec((_RB, D), lambda i: (i, 0)),
          pl.BlockSpec((1, D), lambda i: (0, 0)),
      ],
      out_specs=pl.BlockSpec((_RB, D), lambda i: (i, 0)),
      out_shape=jax.ShapeDtypeStruct((N, D), jnp.float32),
  )(rst, den.reshape(NPAD, 1), y, b.reshape(1, D))


@jax.jit
def _run(x, src2d, dst2d, W0, al0, ar0, b0, W1, al1, ar1, b1):
  h1, el1, er1 = _tc_proj(x, W0, al0, ar0)
  rst1, den1 = _sc_edge_layer(h1.reshape(2 * N, 128), _pad_tab(el1),
                              _pad_tab(er1), src2d, dst2d)
  y1, h2, el2, er2 = _tc_mid(rst1, den1, x, b0, W1, al1, ar1)
  rst2, den2 = _sc_edge_layer(h2.reshape(2 * N, 128), _pad_tab(el2),
                              _pad_tab(er2), src2d, dst2d)
  return _tc_fin(rst2, den2, y1, b1)


def kernel(x, edge_index, W0, al0, ar0, b0, W1, al1, ar1, b1):
  src2d = edge_index[0].astype(jnp.int32).reshape(E // CH, CH)
  dst2d = edge_index[1].astype(jnp.int32).reshape(E // CH, CH)
  return _run(x, src2d, dst2d, W0, al0, ar0, b0, W1, al1, ar1, b1)
